# Initial kernel scaffold; baseline (speedup 1.0000x reference)
import jax
import jax.numpy as jnp
from jax.experimental import pallas as pl


def kernel(x, edge_index, batch, W_rel1, b_rel1, W_root1, W_rel2, b_rel2, W_root2, W_lin, b_lin):
    raise NotImplementedError("write your pallas kernel here")


# speedup vs baseline: 6.2496x; 6.2496x over previous
import functools

import jax
import jax.numpy as jnp
from jax import lax
from jax.experimental import pallas as pl
from jax.experimental.pallas import tpu as pltpu
from jax.experimental.pallas import tpu_sc as plsc

_EPS = 0.001
_HI = jax.lax.Precision.HIGHEST


@functools.cache
def _make_seg_sum(N, D, E):
    CHUNK = 128
    assert E % CHUNK == 0
    NCH = E // CHUNK
    NC, NS = 2, 16
    NW = NC * NS
    RPT = 640
    NPAD = RPT * NS
    assert NPAD >= N and RPT % 8 == 0
    TAIL = N - RPT * (NS - 1)
    assert 0 < TAIL <= RPT and TAIL % 8 == 0
    ZR = 160
    assert RPT % ZR == 0
    rem = NCH % NW

    mesh = plsc.VectorSubcoreMesh(
        core_axis_name="c", subcore_axis_name="s", num_cores=NC, num_subcores=NS
    )

    @functools.partial(
        pl.kernel,
        out_type=jax.ShapeDtypeStruct((NC * N, D), jnp.float32),
        mesh=mesh,
        scratch_types=[
            pltpu.VMEM((CHUNK,), jnp.int32),
            pltpu.VMEM((CHUNK,), jnp.int32),
            pltpu.VMEM((CHUNK, D), jnp.float32),
            pltpu.VMEM((ZR, D), jnp.float32),
            pltpu.VMEM_SHARED((NPAD, D), jnp.float32),
            pltpu.SemaphoreType.DMA,
        ],
    )
    def seg_sum(h_hbm, src_hbm, dst_hbm, out_hbm, src_v, dst_v, rows_v, zb_v,
                acc_sh, sem):
        cid = lax.axis_index("c")
        sid = lax.axis_index("s")
        wid = sid * NC + cid

        def zrow(r, carry):
            for c in range(D // 16):
                zb_v[r, pl.ds(c * 16, 16)] = jnp.zeros((16,), jnp.float32)
            return carry

        lax.fori_loop(0, ZR, zrow, 0)

        def zcopy(i, carry):
            pltpu.sync_copy(zb_v, acc_sh.at[pl.ds(sid * RPT + i * ZR, ZR)])
            return carry

        lax.fori_loop(0, RPT // ZR, zcopy, 0)
        plsc.subcore_barrier()

        nch = NCH // NW + (wid < rem).astype(jnp.int32)

        def body(i, carry):
            base = (wid + i * NW) * CHUNK
            pltpu.sync_copy(src_hbm.at[pl.ds(base, CHUNK)], src_v)
            pltpu.sync_copy(dst_hbm.at[pl.ds(base, CHUNK)], dst_v)
            pltpu.async_copy(h_hbm.at[src_v], rows_v, sem).wait()
            pltpu.sync_copy(rows_v, acc_sh.at[dst_v], add=True)
            return carry

        lax.fori_loop(0, nch, body, 0)
        plsc.subcore_barrier()

        @pl.when(sid < NS - 1)
        def _copy_full():
            pltpu.sync_copy(
                acc_sh.at[pl.ds(sid * RPT, RPT)],
                out_hbm.at[pl.ds(cid * N + sid * RPT, RPT)],
            )

        @pl.when(sid == NS - 1)
        def _copy_tail():
            pltpu.sync_copy(
                acc_sh.at[pl.ds((NS - 1) * RPT, TAIL)],
                out_hbm.at[pl.ds(cid * N + (NS - 1) * RPT, TAIL)],
            )

    return seg_sum


@functools.cache
def _make_layer(N, D, B):
    NB = N // B
    assert N % B == 0

    def body(p0, p1, xb, wrel, brel, wroot, ob):
        agg = p0[...] + p1[...]
        h = (
            jnp.dot(agg, wrel[...], preferred_element_type=jnp.float32)
            + brel[...]
            + jnp.dot(xb[...], wroot[...], preferred_element_type=jnp.float32)
        )
        ob[...] = jnp.maximum(h, 0.0)

    return pl.pallas_call(
        body,
        grid=(NB,),
        in_specs=[
            pl.BlockSpec((B, D), lambda i: (i, 0)),
            pl.BlockSpec((B, D), lambda i: (i + NB, 0)),
            pl.BlockSpec((B, D), lambda i: (i, 0)),
            pl.BlockSpec((D, D), lambda i: (0, 0)),
            pl.BlockSpec((1, D), lambda i: (0, 0)),
            pl.BlockSpec((D, D), lambda i: (0, 0)),
        ],
        out_specs=pl.BlockSpec((B, D), lambda i: (i, 0)),
        out_shape=jax.ShapeDtypeStruct((N, D), jnp.float32),
    )


@functools.cache
def _make_layer2_head(N, D, B):
    NB = N // B
    HB = B // 2
    B2D = D
    assert N % B == 0 and B % 2 == 0

    def body(p0, p1, hb, wrel, brel, wroot, wlin, blin,
             probs_o, out_o, x1_o, x2_o):
        agg = p0[...] + p1[...]
        h2 = (
            jnp.dot(agg, wrel[...], preferred_element_type=jnp.float32)
            + brel[...]
            + jnp.dot(hb[...], wroot[...], preferred_element_type=jnp.float32)
        )
        h2 = jnp.maximum(h2, 0.0)
        h2p = jnp.reshape(h2, (HB, 2 * B2D))
        x1 = h2p[:, :B2D]
        x2 = h2p[:, B2D:]
        out = jnp.sqrt((x1 - x2) ** 2 + _EPS)
        logit = jnp.dot(out, wlin[...], preferred_element_type=jnp.float32)
        probs_o[...] = 1.0 / (1.0 + jnp.exp(-(logit + blin[...])))
        out_o[...] = out
        x1_o[...] = x1
        x2_o[...] = x2

    return pl.pallas_call(
        body,
        grid=(NB,),
        in_specs=[
            pl.BlockSpec((B, D), lambda i: (i, 0)),
            pl.BlockSpec((B, D), lambda i: (i + NB, 0)),
            pl.BlockSpec((B, D), lambda i: (i, 0)),
            pl.BlockSpec((D, D), lambda i: (0, 0)),
            pl.BlockSpec((1, D), lambda i: (0, 0)),
            pl.BlockSpec((D, D), lambda i: (0, 0)),
            pl.BlockSpec((D, 1), lambda i: (0, 0)),
            pl.BlockSpec((1, 1), lambda i: (0, 0)),
        ],
        out_specs=[
            pl.BlockSpec((HB, 1), lambda i: (i, 0)),
            pl.BlockSpec((HB, D), lambda i: (i, 0)),
            pl.BlockSpec((HB, D), lambda i: (i, 0)),
            pl.BlockSpec((HB, D), lambda i: (i, 0)),
        ],
        out_shape=[
            jax.ShapeDtypeStruct((N // 2, 1), jnp.float32),
            jax.ShapeDtypeStruct((N // 2, D), jnp.float32),
            jax.ShapeDtypeStruct((N // 2, D), jnp.float32),
            jax.ShapeDtypeStruct((N // 2, D), jnp.float32),
        ],
    )


def kernel(x, edge_index, batch, W_rel1, b_rel1, W_root1, W_rel2, b_rel2,
           W_root2, W_lin, b_lin):
    N, D = x.shape
    E = edge_index.shape[1]
    src = edge_index[0]
    dst = edge_index[1]

    seg_sum = _make_seg_sum(N, D, E)
    layer1 = _make_layer(N, D, 1000)
    layer2 = _make_layer2_head(N, D, 400)

    brel1 = b_rel1.reshape(1, D)
    brel2 = b_rel2.reshape(1, D)
    wlin = W_lin.reshape(D, 1)
    blin = b_lin.reshape(1, 1)

    part1 = seg_sum(x, src, dst)
    h1 = layer1(part1, part1, x, W_rel1, brel1, W_root1)
    part2 = seg_sum(h1, src, dst)
    probs, out, x1, x2 = layer2(part2, part2, h1, W_rel2, brel2, W_root2,
                                wlin, blin)
    return (probs, out, x1, x2)
